# Initial kernel scaffold; baseline (speedup 1.0000x reference)
import jax
import jax.numpy as jnp
from jax.experimental import pallas as pl


def kernel(xyz, signal, weight, biases):
    raise NotImplementedError("write your pallas kernel here")


# speedup vs baseline: 15.6891x; 15.6891x over previous
import math

import jax
import jax.numpy as jnp
from jax.experimental import pallas as pl
from jax.experimental.pallas import tpu as pltpu

_L_MAX = 2
_NR = 4
_KERNEL_RADIUS = 2.0
_PATCH_K = 64
_SIGMA = _KERNEL_RADIUS / (_NR - 1)
_INV_2SIG2 = 1.0 / (2.0 * _SIGMA * _SIGMA)

_C0 = 0.5 * math.sqrt(1.0 / math.pi)
_C1 = math.sqrt(3.0 / (4.0 * math.pi))
_C2A = 0.5 * math.sqrt(15.0 / math.pi)
_C2B = 0.25 * math.sqrt(5.0 / math.pi)
_C2C = 0.25 * math.sqrt(15.0 / math.pi)

_VB = 128
_NB = 512


def _body(xv_ref, xn_ref, sig_ref, wf_ref, b_ref, out_ref, d_scr):
    N = xn_ref.shape[2]
    C = sig_ref.shape[2]
    xv = xv_ref[0]
    xn = xn_ref[0]
    sig = sig_ref[0]

    vx, vy, vz = xv[:, 0:1], xv[:, 1:2], xv[:, 2:3]
    nx, ny, nz = xn[0:1, :], xn[1:2, :], xn[2:3, :]
    nv = vx * vx + vy * vy + vz * vz
    nn = nx * nx + ny * ny + nz * nz
    def _rb(t):
        return t.astype(jnp.bfloat16).astype(jnp.float32)

    m = _rb(vx) * _rb(nx) + _rb(vy) * _rb(ny) + _rb(vz) * _rb(nz)
    d_scr[...] = nv - 2.0 * m + nn
    D = d_scr[...]

    ub = jax.lax.bitcast_convert_type(jnp.maximum(D, 0.0), jnp.int32)

    def bit_step(i, P):
        bit = 30 - i
        Q = P | (jnp.int32(1) << bit)
        cnt = jnp.sum((ub < Q).astype(jnp.int32), axis=1, keepdims=True)
        return jnp.where(cnt < _PATCH_K, Q, P)

    P = jax.lax.fori_loop(0, 31, bit_step,
                          jnp.zeros((xv.shape[0], 1), jnp.int32))
    mask = (ub <= P).astype(jnp.float32)

    dist = jnp.sqrt(jnp.maximum(D, 1e-4))
    rad0 = jnp.exp(-(dist * dist) * _INV_2SIG2)
    y_w = jnp.sum(mask * rad0, axis=1, keepdims=True) * _C0
    g = mask * (1.0 / (y_w + 1e-6))

    VB = xv.shape[0]
    acc = jnp.zeros((_NR * 9 * VB, C), jnp.float32)
    for c in range(N // _NB):
        sl = slice(c * _NB, (c + 1) * _NB)
        distc = dist[:, sl]
        gc = g[:, sl]
        dx = nx[:, sl] - vx
        dy = ny[:, sl] - vy
        dz = nz[:, sl] - vz
        inv = jax.lax.rsqrt(dx * dx + dy * dy + dz * dz + 1e-8)
        ux, uy, uz = dx * inv, dy * inv, dz * inv
        Ys = (jnp.full_like(ux, _C0), _C1 * uy, _C1 * uz, _C1 * ux,
              _C2A * ux * uy, _C2A * uy * uz, _C2B * (3.0 * uz * uz - 1.0),
              _C2A * ux * uz, _C2C * (ux * ux - uy * uy))
        slabs = []
        for r in range(_NR):
            rr = distc - r * (_KERNEL_RADIUS / (_NR - 1))
            grc = gc * jnp.exp(-(rr * rr) * _INV_2SIG2)
            for s in range(9):
                slabs.append(grc * Ys[s])
        A = jnp.concatenate(slabs, axis=0)
        acc = acc + jnp.dot(A.astype(jnp.bfloat16),
                            sig[sl, :].astype(jnp.bfloat16),
                            preferred_element_type=jnp.float32)

    sq = acc * acc
    pieces = []
    for r in range(_NR):
        base = r * 9 * VB
        p0 = sq[base:base + VB]
        p1 = (sq[base + VB:base + 2 * VB]
              + sq[base + 2 * VB:base + 3 * VB]
              + sq[base + 3 * VB:base + 4 * VB])
        p2 = (sq[base + 4 * VB:base + 5 * VB]
              + sq[base + 5 * VB:base + 6 * VB]
              + sq[base + 6 * VB:base + 7 * VB]
              + sq[base + 7 * VB:base + 8 * VB]
              + sq[base + 8 * VB:base + 9 * VB])
        pieces += [p0, p1, p2]
    cat = jnp.concatenate(pieces, axis=1)
    cat = jnp.sqrt(jnp.maximum(cat, 1e-4))
    out = jnp.dot(cat.astype(jnp.bfloat16),
                  wf_ref[...].astype(jnp.bfloat16),
                  preferred_element_type=jnp.float32) + b_ref[...]
    out_ref[0] = out


def _sphconv(xyz, signal, weight, biases, interpret=False):
    B, N, _ = xyz.shape
    C = signal.shape[2]
    CO = weight.shape[0]
    xt = jnp.swapaxes(xyz, 1, 2)
    pad_n = jnp.zeros((B, 5, N), jnp.float32)
    xn = jnp.concatenate([xt, pad_n], axis=1)
    pad_v = jnp.zeros((B, N, 5), jnp.float32)
    xv = jnp.concatenate([xyz, pad_v], axis=2)
    wf = jnp.transpose(weight, (2, 3, 1, 0)).reshape(_NR * 3 * C, CO)
    b2 = biases.reshape(1, CO)

    grid = (B, N // _VB)
    return pl.pallas_call(
        _body,
        grid=grid,
        in_specs=[
            pl.BlockSpec((1, _VB, 8), lambda b, v: (b, v, 0)),
            pl.BlockSpec((1, 8, N), lambda b, v: (b, 0, 0)),
            pl.BlockSpec((1, N, C), lambda b, v: (b, 0, 0)),
            pl.BlockSpec((_NR * 3 * C, CO), lambda b, v: (0, 0)),
            pl.BlockSpec((1, CO), lambda b, v: (0, 0)),
        ],
        out_specs=pl.BlockSpec((1, _VB, CO), lambda b, v: (b, v, 0)),
        out_shape=jax.ShapeDtypeStruct((B, N, CO), jnp.float32),
        scratch_shapes=[pltpu.VMEM((_VB, N), jnp.float32)],
        compiler_params=pltpu.CompilerParams(
            dimension_semantics=("parallel", "parallel")),
        interpret=interpret,
    )(xv, xn, sig_f32(signal), wf, b2)


def sig_f32(signal):
    return signal.astype(jnp.float32)


def kernel(xyz, signal, weight, biases):
    return _sphconv(xyz, signal, weight, biases)
